# baseline (speedup 1.0000x reference)
import jax
import jax.numpy as jnp
from jax import lax
from jax.experimental import pallas as pl
from jax.experimental.pallas import tpu as pltpu
from jax.experimental.pallas import tpu_sc as plsc

N = 10000
E = 160000
NUM_RELS = 8
NUM_BASES = 4
HID = 512
IN_DIM = 3072
OUT_DIM = 64

MBLK = 1000

NC, NS, LANES = 2, 16, 16
HALF = HID // NC
RPT = 624
ACC_ROWS = 352
P1_A = 272
P1_B = 288
CH = 2000
NCHUNK = E // CH
KBLK = 64
PBUF = CH + 2 * KBLK + 16


def _wcomb_body(c_ref, b_ref, o_ref):
    o_ref[...] = jnp.dot(c_ref[...], b_ref[...], preferred_element_type=jnp.float32)


def _weight_combine(coef, basis):
    out = pl.pallas_call(
        _wcomb_body,
        out_shape=jax.ShapeDtypeStruct((NUM_RELS, HID * HID), jnp.float32),
    )(coef, basis.reshape(NUM_BASES, HID * HID))
    return out.reshape(NUM_RELS, HID, HID)


def _fnn_body(h_ref, w_ref, b_ref, o_ref):
    acc = jnp.dot(h_ref[...], w_ref[...], preferred_element_type=jnp.float32)
    o_ref[...] = jnp.maximum(acc + b_ref[...], 0.0)


def _fnn(h, W_fnn, b_fnn):
    m = h.shape[0]
    return pl.pallas_call(
        _fnn_body,
        grid=(m // MBLK,),
        in_specs=[
            pl.BlockSpec((MBLK, IN_DIM), lambda i: (i, 0)),
            pl.BlockSpec((IN_DIM, 256), lambda i: (0, 0)),
            pl.BlockSpec((1, 256), lambda i: (0, 0)),
        ],
        out_specs=pl.BlockSpec((MBLK, 256), lambda i: (i, 0)),
        out_shape=jax.ShapeDtypeStruct((m, 256), jnp.float32),
    )(h, W_fnn, b_fnn.reshape(1, 256))


def _rel_mm_body(x_ref, w_ref, o_ref):
    o_ref[0] = jnp.dot(x_ref[...], w_ref[0], preferred_element_type=jnp.float32)


def _rel_matmul(x, weight):
    m, k = x.shape
    return pl.pallas_call(
        _rel_mm_body,
        grid=(NUM_RELS, m // MBLK),
        in_specs=[
            pl.BlockSpec((MBLK, k), lambda r, i: (i, 0)),
            pl.BlockSpec((1, k, HID), lambda r, i: (r, 0, 0)),
        ],
        out_specs=pl.BlockSpec((1, MBLK, HID), lambda r, i: (r, i, 0)),
        out_shape=jax.ShapeDtypeStruct((NUM_RELS, m, HID), jnp.float32),
    )(x, weight)


def _rel_mm2_body(a0_ref, a1_ref, b_ref, w_ref, o_ref):
    x = jnp.concatenate([a0_ref[...], a1_ref[...]], axis=-1)
    x = jnp.maximum(x + b_ref[...], 0.0)
    o_ref[0] = jnp.dot(x, w_ref[0], preferred_element_type=jnp.float32)


def _rel_matmul_fused(a0, a1, h_bias, weight):
    m = a0.shape[0]
    return pl.pallas_call(
        _rel_mm2_body,
        grid=(NUM_RELS, m // MBLK),
        in_specs=[
            pl.BlockSpec((MBLK, HALF), lambda r, i: (i, 0)),
            pl.BlockSpec((MBLK, HALF), lambda r, i: (i, 0)),
            pl.BlockSpec((1, HID), lambda r, i: (0, 0)),
            pl.BlockSpec((1, HID, HID), lambda r, i: (r, 0, 0)),
        ],
        out_specs=pl.BlockSpec((1, MBLK, HID), lambda r, i: (r, i, 0)),
        out_shape=jax.ShapeDtypeStruct((NUM_RELS, m, HID), jnp.float32),
    )(a0, a1, h_bias.reshape(1, HID), weight)


def _out_body(a0_ref, a1_ref, hb_ref, w_ref, b_ref, o_ref):
    x = jnp.concatenate([a0_ref[...], a1_ref[...]], axis=-1)
    x = jnp.maximum(x + hb_ref[...], 0.0)
    logits = jnp.dot(x, w_ref[...], preferred_element_type=jnp.float32)
    logits = logits + b_ref[...]
    mx = jnp.max(logits, axis=-1, keepdims=True)
    e = jnp.exp(logits - mx)
    o_ref[...] = e / jnp.sum(e, axis=-1, keepdims=True)


def _out_proj(a0, a1, h_bias, W_out, b_out):
    m = a0.shape[0]
    return pl.pallas_call(
        _out_body,
        grid=(m // MBLK,),
        in_specs=[
            pl.BlockSpec((MBLK, HALF), lambda i: (i, 0)),
            pl.BlockSpec((MBLK, HALF), lambda i: (i, 0)),
            pl.BlockSpec((1, HID), lambda i: (0, 0)),
            pl.BlockSpec((HID, OUT_DIM), lambda i: (0, 0)),
            pl.BlockSpec((1, OUT_DIM), lambda i: (0, 0)),
        ],
        out_specs=pl.BlockSpec((MBLK, OUT_DIM), lambda i: (i, 0)),
        out_shape=jax.ShapeDtypeStruct((m, OUT_DIM), jnp.float32),
    )(a0, a1, h_bias.reshape(1, HID), W_out, b_out.reshape(1, OUT_DIM))


def _sc_body(table, base_idx, dstg, nrm, out,
             dstS, gidxS, nrmS, pd, pg, pn, gbuf, acc, sem):
    c = lax.axis_index("c")
    s = lax.axis_index("s")

    zero16i = jnp.zeros((16,), jnp.int32)
    zero16f = jnp.zeros((16,), jnp.float32)

    tile_lo = s * RPT

    def blk(j, _):
        o = j * KBLK
        pltpu.async_copy(table.at[pg.at[pl.ds(o, KBLK)]], gbuf, sem).wait()

        def srow(t, _2):
            nv = pn[pl.ds(o + t, 16)]
            dv = pd[pl.ds(o + t, 16)]
            bv = jnp.full((16,), nv[0], dtype=jnp.float32)
            drow = dv[0]
            for k in range(HALF // 16):
                plsc.addupdate(acc.at[drow, pl.ds(k * 16, 16)],
                               gbuf[t, pl.ds(k * 16, 16)] * bv)
            return 0

        lax.fori_loop(0, KBLK, srow, 0)
        return 0

    for p in range(2):
        if p == 0:
            lo = tile_lo
            hi = tile_lo + ACC_ROWS
        else:
            lo = tile_lo + ACC_ROWS
            hi = tile_lo + jnp.where(s == 15, 640, RPT).astype(jnp.int32)

        def zrow(i, _):
            for k in range(HALF // 16):
                acc[i, pl.ds(k * 16, 16)] = zero16f
            return 0

        lax.fori_loop(0, ACC_ROWS, zrow, 0)

        def chunk(ch, cnt):
            pltpu.sync_copy(dstg.at[pl.ds(ch * CH, CH)], dstS.at[pl.ds(0, CH)])
            pltpu.sync_copy(base_idx.at[pl.ds(ch * CH, CH)],
                            gidxS.at[pl.ds(0, CH)])
            pltpu.sync_copy(nrm.at[pl.ds(ch * CH, CH)], nrmS.at[pl.ds(0, CH)])

            def comp(i, cn):
                dvv = dstS[pl.ds(i * 16, 16)]
                gvv = gidxS[pl.ds(i * 16, 16)]
                nvv = nrmS[pl.ds(i * 16, 16)]
                m = (dvv >= lo) & (dvv < hi)
                delta = plsc.all_reduce_population_count(m)[0]
                plsc.store_compressed(pd.at[pl.ds(cn, 16)], dvv - lo, mask=m)
                plsc.store_compressed(pg.at[pl.ds(cn, 16)], gvv + c, mask=m)
                plsc.store_compressed(pn.at[pl.ds(cn, 16)], nvv, mask=m)
                return cn + delta

            cnt = lax.fori_loop(0, CH // 16, comp, cnt)
            nb = cnt // KBLK
            lax.fori_loop(0, nb, blk, 0)
            off = nb * KBLK
            for k in range(KBLK // 16):
                vd = pd[pl.ds(off + k * 16, 16)]
                vg = pg[pl.ds(off + k * 16, 16)]
                vn = pn[pl.ds(off + k * 16, 16)]
                pd[pl.ds(k * 16, 16)] = vd
                pg[pl.ds(k * 16, 16)] = vg
                pn[pl.ds(k * 16, 16)] = vn
            return cnt - off

        cnt = lax.fori_loop(0, NCHUNK, chunk, jnp.int32(0))

        for k in range(KBLK // 16):
            pd[pl.ds(cnt + k * 16, 16)] = zero16i
            pg[pl.ds(cnt + k * 16, 16)] = zero16i
            pn[pl.ds(cnt + k * 16, 16)] = zero16f

        @pl.when(cnt > 0)
        def _():
            blk(0, 0)

        obase = c * N + lo
        if p == 0:
            pltpu.sync_copy(acc.at[pl.ds(0, ACC_ROWS)],
                            out.at[pl.ds(obase, ACC_ROWS)])
        else:
            @pl.when(s < 15)
            def _():
                pltpu.sync_copy(acc.at[pl.ds(0, P1_A)],
                                out.at[pl.ds(obase, P1_A)])

            @pl.when(s == 15)
            def _():
                pltpu.sync_copy(acc.at[pl.ds(0, P1_B)],
                                out.at[pl.ds(obase, P1_B)])


def _sc_aggregate(table_flat, base_idx, dstv, nrmv):
    mesh = plsc.VectorSubcoreMesh(core_axis_name="c", subcore_axis_name="s",
                                  num_cores=NC, num_subcores=NS)
    f = pl.kernel(
        _sc_body,
        out_type=jax.ShapeDtypeStruct((NC * N, HALF), jnp.float32),
        mesh=mesh,
        compiler_params=pltpu.CompilerParams(needs_layout_passes=False),
        scratch_types=[
            pltpu.VMEM((CH,), jnp.int32),
            pltpu.VMEM((CH,), jnp.int32),
            pltpu.VMEM((CH,), jnp.float32),
            pltpu.VMEM((PBUF,), jnp.int32),
            pltpu.VMEM((PBUF,), jnp.int32),
            pltpu.VMEM((PBUF,), jnp.float32),
            pltpu.VMEM((KBLK, HALF), jnp.float32),
            pltpu.VMEM((ACC_ROWS, HALF), jnp.float32),
            pltpu.SemaphoreType.DMA,
        ],
    )
    return f(table_flat, base_idx, dstv, nrmv).reshape(NC, N, HALF)


def kernel(g, h, r, norm, W_fnn, b_fnn, basis, coef, h_bias, W_out, b_out):
    src = g[0].astype(jnp.int32)
    dst = g[1].astype(jnp.int32)
    rr = r.reshape(-1).astype(jnp.int32)
    nrmv = norm.reshape(-1)
    base_idx = rr * (2 * N) + src * 2

    weight = _weight_combine(coef, basis)
    x1 = _fnn(h, W_fnn, b_fnn)

    Hr = _rel_matmul(x1, weight[:, :256, :])
    agg = _sc_aggregate(Hr.reshape(2 * NUM_RELS * N, HALF), base_idx, dst, nrmv)

    Hr = _rel_matmul_fused(agg[0], agg[1], h_bias, weight)
    agg = _sc_aggregate(Hr.reshape(2 * NUM_RELS * N, HALF), base_idx, dst, nrmv)

    return _out_proj(agg[0], agg[1], h_bias, W_out, b_out)


# speedup vs baseline: 1.1945x; 1.1945x over previous
import jax
import jax.numpy as jnp
from jax import lax
from jax.experimental import pallas as pl
from jax.experimental.pallas import tpu as pltpu
from jax.experimental.pallas import tpu_sc as plsc

N = 10000
E = 160000
NUM_RELS = 8
NUM_BASES = 4
HID = 512
IN_DIM = 3072
OUT_DIM = 64

MBLK = 1000

NC, NS, LANES = 2, 16, 16
HALF = HID // NC
RPT = 624
ACC_ROWS = 320
P1_A = 304
P1_B = 320
CH = 2000
NCHUNK = E // CH
KBLK = 48
PBUF = CH + 2 * KBLK + 16


def _wcomb_body(c_ref, b_ref, o_ref):
    w = jnp.dot(c_ref[...], b_ref[...], preferred_element_type=jnp.float32)
    o_ref[...] = w.astype(jnp.bfloat16)


def _weight_combine(coef, basis):
    out = pl.pallas_call(
        _wcomb_body,
        out_shape=jax.ShapeDtypeStruct((NUM_RELS, HID * HID), jnp.bfloat16),
    )(coef, basis.reshape(NUM_BASES, HID * HID))
    return out.reshape(NUM_RELS, HID, HID)


def _fnn_body(h_ref, w_ref, b_ref, o_ref):
    acc = jnp.dot(h_ref[...].astype(jnp.bfloat16),
                  w_ref[...].astype(jnp.bfloat16),
                  preferred_element_type=jnp.float32)
    o_ref[...] = jnp.maximum(acc + b_ref[...], 0.0)


def _fnn(h, W_fnn, b_fnn):
    m = h.shape[0]
    return pl.pallas_call(
        _fnn_body,
        grid=(m // MBLK,),
        in_specs=[
            pl.BlockSpec((MBLK, IN_DIM), lambda i: (i, 0)),
            pl.BlockSpec((IN_DIM, 256), lambda i: (0, 0)),
            pl.BlockSpec((1, 256), lambda i: (0, 0)),
        ],
        out_specs=pl.BlockSpec((MBLK, 256), lambda i: (i, 0)),
        out_shape=jax.ShapeDtypeStruct((m, 256), jnp.float32),
    )(h, W_fnn, b_fnn.reshape(1, 256))


def _rel_mm_body(x_ref, w_ref, o_ref):
    o_ref[0] = jnp.dot(x_ref[...].astype(jnp.bfloat16), w_ref[0],
                       preferred_element_type=jnp.float32)


def _rel_matmul(x, weight):
    m, k = x.shape
    return pl.pallas_call(
        _rel_mm_body,
        grid=(NUM_RELS, m // MBLK),
        in_specs=[
            pl.BlockSpec((MBLK, k), lambda r, i: (i, 0)),
            pl.BlockSpec((1, k, HID), lambda r, i: (r, 0, 0)),
        ],
        out_specs=pl.BlockSpec((1, MBLK, HID), lambda r, i: (r, i, 0)),
        out_shape=jax.ShapeDtypeStruct((NUM_RELS, m, HID), jnp.float32),
    )(x, weight)


def _rel_mm2_body(a0_ref, a1_ref, b_ref, w_ref, o_ref):
    x = jnp.concatenate([a0_ref[...], a1_ref[...]], axis=-1)
    x = jnp.maximum(x + b_ref[...], 0.0)
    o_ref[0] = jnp.dot(x.astype(jnp.bfloat16), w_ref[0],
                       preferred_element_type=jnp.float32)


def _rel_matmul_fused(a0, a1, h_bias, weight):
    m = a0.shape[0]
    return pl.pallas_call(
        _rel_mm2_body,
        grid=(NUM_RELS, m // MBLK),
        in_specs=[
            pl.BlockSpec((MBLK, HALF), lambda r, i: (i, 0)),
            pl.BlockSpec((MBLK, HALF), lambda r, i: (i, 0)),
            pl.BlockSpec((1, HID), lambda r, i: (0, 0)),
            pl.BlockSpec((1, HID, HID), lambda r, i: (r, 0, 0)),
        ],
        out_specs=pl.BlockSpec((1, MBLK, HID), lambda r, i: (r, i, 0)),
        out_shape=jax.ShapeDtypeStruct((NUM_RELS, m, HID), jnp.float32),
    )(a0, a1, h_bias.reshape(1, HID), weight)


def _out_body(a0_ref, a1_ref, hb_ref, w_ref, b_ref, o_ref):
    x = jnp.concatenate([a0_ref[...], a1_ref[...]], axis=-1)
    x = jnp.maximum(x + hb_ref[...], 0.0)
    logits = jnp.dot(x, w_ref[...], preferred_element_type=jnp.float32)
    logits = logits + b_ref[...]
    mx = jnp.max(logits, axis=-1, keepdims=True)
    e = jnp.exp(logits - mx)
    o_ref[...] = e / jnp.sum(e, axis=-1, keepdims=True)


def _out_proj(a0, a1, h_bias, W_out, b_out):
    m = a0.shape[0]
    return pl.pallas_call(
        _out_body,
        grid=(m // MBLK,),
        in_specs=[
            pl.BlockSpec((MBLK, HALF), lambda i: (i, 0)),
            pl.BlockSpec((MBLK, HALF), lambda i: (i, 0)),
            pl.BlockSpec((1, HID), lambda i: (0, 0)),
            pl.BlockSpec((HID, OUT_DIM), lambda i: (0, 0)),
            pl.BlockSpec((1, OUT_DIM), lambda i: (0, 0)),
        ],
        out_specs=pl.BlockSpec((MBLK, OUT_DIM), lambda i: (i, 0)),
        out_shape=jax.ShapeDtypeStruct((m, OUT_DIM), jnp.float32),
    )(a0, a1, h_bias.reshape(1, HID), W_out, b_out.reshape(1, OUT_DIM))


def _sc_body(table, base_idx, dstg, nrm, out,
             dstS, gidxS, nrmS, pd, pg, pn, gbuf, acc, ssem, gsem):
    c = lax.axis_index("c")
    s = lax.axis_index("s")

    zero16i = jnp.zeros((16,), jnp.int32)
    zero16f = jnp.zeros((16,), jnp.float32)

    tile_lo = s * RPT

    def stage_start(ch, slot):
        sb = pl.ds(slot * CH, CH)
        pltpu.async_copy(dstg.at[pl.ds(ch * CH, CH)], dstS.at[sb], ssem)
        pltpu.async_copy(base_idx.at[pl.ds(ch * CH, CH)], gidxS.at[sb], ssem)
        pltpu.async_copy(nrm.at[pl.ds(ch * CH, CH)], nrmS.at[sb], ssem)

    def stage_wait(slot):
        sb = pl.ds(slot * CH, CH)
        pltpu.make_async_copy(dstg.at[pl.ds(0, CH)], dstS.at[sb], ssem).wait()
        pltpu.make_async_copy(base_idx.at[pl.ds(0, CH)], gidxS.at[sb],
                              ssem).wait()
        pltpu.make_async_copy(nrm.at[pl.ds(0, CH)], nrmS.at[sb], ssem).wait()

    def gather_start(j, slot):
        pltpu.async_copy(table.at[pg.at[pl.ds(j * KBLK, KBLK)]],
                         gbuf.at[pl.ds(slot * KBLK, KBLK)], gsem)

    def gather_wait(slot):
        pltpu.make_async_copy(table.at[pl.ds(0, KBLK)],
                              gbuf.at[pl.ds(slot * KBLK, KBLK)], gsem).wait()

    def proc(j, _):
        o = j * KBLK
        slot = j % 2
        gather_wait(slot)

        def srow(t, _2):
            nv = pn[pl.ds(o + t, 16)]
            dv = pd[pl.ds(o + t, 16)]
            bv = jnp.full((16,), nv[0], dtype=jnp.float32)
            drow = dv[0]
            for k in range(HALF // 16):
                plsc.addupdate(acc.at[drow, pl.ds(k * 16, 16)],
                               gbuf[slot * KBLK + t, pl.ds(k * 16, 16)] * bv)
            return 0

        lax.fori_loop(0, KBLK, srow, 0)
        return 0

    for p in range(2):
        if p == 0:
            lo = tile_lo
            hi = tile_lo + ACC_ROWS
        else:
            lo = tile_lo + ACC_ROWS
            hi = tile_lo + jnp.where(s == 15, 640, RPT).astype(jnp.int32)

        def zrow(i, _):
            for k in range(HALF // 16):
                acc[i, pl.ds(k * 16, 16)] = zero16f
            return 0

        lax.fori_loop(0, ACC_ROWS, zrow, 0)

        stage_start(0, 0)

        def chunk(ch, cnt):
            slot = ch % 2
            stage_wait(slot)

            @pl.when(ch + 1 < NCHUNK)
            def _():
                stage_start(ch + 1, 1 - slot)

            def comp(i, cn):
                dvv = dstS[pl.ds(slot * CH + i * 16, 16)]
                gvv = gidxS[pl.ds(slot * CH + i * 16, 16)]
                nvv = nrmS[pl.ds(slot * CH + i * 16, 16)]
                m = (dvv >= lo) & (dvv < hi)
                delta = plsc.all_reduce_population_count(m)[0]
                plsc.store_compressed(pd.at[pl.ds(cn, 16)], dvv - lo, mask=m)
                plsc.store_compressed(pg.at[pl.ds(cn, 16)], gvv + c, mask=m)
                plsc.store_compressed(pn.at[pl.ds(cn, 16)], nvv, mask=m)
                return cn + delta

            cnt = lax.fori_loop(0, CH // 16, comp, cnt)
            nb = cnt // KBLK

            @pl.when(nb > 0)
            def _():
                gather_start(0, 0)

            @pl.when(nb > 1)
            def _():
                gather_start(1, 1)

            def proc2(j, _):
                proc(j, 0)

                @pl.when(j + 2 < nb)
                def _():
                    gather_start(j + 2, j % 2)
                return 0

            lax.fori_loop(0, nb, proc2, 0)
            off = nb * KBLK
            for k in range(KBLK // 16):
                vd = pd[pl.ds(off + k * 16, 16)]
                vg = pg[pl.ds(off + k * 16, 16)]
                vn = pn[pl.ds(off + k * 16, 16)]
                pd[pl.ds(k * 16, 16)] = vd
                pg[pl.ds(k * 16, 16)] = vg
                pn[pl.ds(k * 16, 16)] = vn
            return cnt - off

        cnt = lax.fori_loop(0, NCHUNK, chunk, jnp.int32(0))

        for k in range(KBLK // 16):
            pd[pl.ds(cnt + k * 16, 16)] = zero16i
            pg[pl.ds(cnt + k * 16, 16)] = zero16i
            pn[pl.ds(cnt + k * 16, 16)] = zero16f

        @pl.when(cnt > 0)
        def _():
            gather_start(0, 0)
            proc(0, 0)

        obase = c * N + lo
        if p == 0:
            pltpu.sync_copy(acc.at[pl.ds(0, ACC_ROWS)],
                            out.at[pl.ds(obase, ACC_ROWS)])
        else:
            @pl.when(s < 15)
            def _():
                pltpu.sync_copy(acc.at[pl.ds(0, P1_A)],
                                out.at[pl.ds(obase, P1_A)])

            @pl.when(s == 15)
            def _():
                pltpu.sync_copy(acc.at[pl.ds(0, P1_B)],
                                out.at[pl.ds(obase, P1_B)])


def _sc_aggregate(table_flat, base_idx, dstv, nrmv):
    mesh = plsc.VectorSubcoreMesh(core_axis_name="c", subcore_axis_name="s",
                                  num_cores=NC, num_subcores=NS)
    f = pl.kernel(
        _sc_body,
        out_type=jax.ShapeDtypeStruct((NC * N, HALF), jnp.float32),
        mesh=mesh,
        compiler_params=pltpu.CompilerParams(needs_layout_passes=False),
        scratch_types=[
            pltpu.VMEM((2 * CH,), jnp.int32),
            pltpu.VMEM((2 * CH,), jnp.int32),
            pltpu.VMEM((2 * CH,), jnp.float32),
            pltpu.VMEM((PBUF,), jnp.int32),
            pltpu.VMEM((PBUF,), jnp.int32),
            pltpu.VMEM((PBUF,), jnp.float32),
            pltpu.VMEM((2 * KBLK, HALF), jnp.float32),
            pltpu.VMEM((ACC_ROWS, HALF), jnp.float32),
            pltpu.SemaphoreType.DMA,
            pltpu.SemaphoreType.DMA,
        ],
    )
    return f(table_flat, base_idx, dstv, nrmv).reshape(NC, N, HALF)


def kernel(g, h, r, norm, W_fnn, b_fnn, basis, coef, h_bias, W_out, b_out):
    src = g[0].astype(jnp.int32)
    dst = g[1].astype(jnp.int32)
    rr = r.reshape(-1).astype(jnp.int32)
    nrmv = norm.reshape(-1)
    base_idx = rr * (2 * N) + src * 2

    weight = _weight_combine(coef, basis)
    x1 = _fnn(h, W_fnn, b_fnn)

    Hr = _rel_matmul(x1, weight[:, :256, :])
    agg = _sc_aggregate(Hr.reshape(2 * NUM_RELS * N, HALF), base_idx, dst, nrmv)

    Hr = _rel_matmul_fused(agg[0], agg[1], h_bias, weight)
    agg = _sc_aggregate(Hr.reshape(2 * NUM_RELS * N, HALF), base_idx, dst, nrmv)

    return _out_proj(agg[0], agg[1], h_bias, W_out, b_out)


# speedup vs baseline: 1.2558x; 1.0513x over previous
import jax
import jax.numpy as jnp
from jax import lax
from jax.experimental import pallas as pl
from jax.experimental.pallas import tpu as pltpu
from jax.experimental.pallas import tpu_sc as plsc

N = 10000
E = 160000
NUM_RELS = 8
NUM_BASES = 4
HID = 512
IN_DIM = 3072
OUT_DIM = 64

MBLK = 1000

NC, NS, LANES = 2, 16, 16
HALF = HID // NC
N2 = 10240
RPT = 640
ACC_ROWS = 320
CH = 2000
NCHUNK = E // CH
KBLK = 48
PBUF = CH + 2 * KBLK + 16


def _wcomb_body(c_ref, b_ref, o_ref):
    w = jnp.dot(c_ref[...], b_ref[...], preferred_element_type=jnp.float32)
    o_ref[...] = w.astype(jnp.bfloat16)


def _weight_combine(coef, basis):
    out = pl.pallas_call(
        _wcomb_body,
        out_shape=jax.ShapeDtypeStruct((NUM_RELS, HID * HID), jnp.bfloat16),
    )(coef, basis.reshape(NUM_BASES, HID * HID))
    return out.reshape(NUM_RELS, HID, HID)


def _fnn_body(h_ref, w_ref, b_ref, o_ref):
    acc = jnp.dot(h_ref[...].astype(jnp.bfloat16),
                  w_ref[...].astype(jnp.bfloat16),
                  preferred_element_type=jnp.float32)
    o_ref[...] = jnp.maximum(acc + b_ref[...], 0.0)


def _fnn(h, W_fnn, b_fnn):
    m = h.shape[0]
    return pl.pallas_call(
        _fnn_body,
        grid=(m // MBLK,),
        in_specs=[
            pl.BlockSpec((MBLK, IN_DIM), lambda i: (i, 0)),
            pl.BlockSpec((IN_DIM, 256), lambda i: (0, 0)),
            pl.BlockSpec((1, 256), lambda i: (0, 0)),
        ],
        out_specs=pl.BlockSpec((MBLK, 256), lambda i: (i, 0)),
        out_shape=jax.ShapeDtypeStruct((m, 256), jnp.float32),
    )(h, W_fnn, b_fnn.reshape(1, 256))


def _rel_mm_body(x_ref, w_ref, o_ref):
    o_ref[0] = jnp.dot(x_ref[...].astype(jnp.bfloat16), w_ref[0],
                       preferred_element_type=jnp.float32)


def _rel_matmul(x, weight):
    m, k = x.shape
    return pl.pallas_call(
        _rel_mm_body,
        grid=(NUM_RELS, m // MBLK),
        in_specs=[
            pl.BlockSpec((MBLK, k), lambda r, i: (i, 0)),
            pl.BlockSpec((1, k, HID), lambda r, i: (r, 0, 0)),
        ],
        out_specs=pl.BlockSpec((1, MBLK, HID), lambda r, i: (r, i, 0)),
        out_shape=jax.ShapeDtypeStruct((NUM_RELS, m, HID), jnp.float32),
    )(x, weight)


def _rel_mm2_body(a0_ref, a1_ref, b_ref, w_ref, o_ref):
    x = jnp.concatenate([a0_ref[...], a1_ref[...]], axis=-1)
    x = jnp.maximum(x + b_ref[...], 0.0)
    o_ref[0] = jnp.dot(x.astype(jnp.bfloat16), w_ref[0],
                       preferred_element_type=jnp.float32)


def _rel_matmul_fused(a0, a1, h_bias, weight):
    m = a0.shape[0]
    return pl.pallas_call(
        _rel_mm2_body,
        grid=(NUM_RELS, m // MBLK),
        in_specs=[
            pl.BlockSpec((MBLK, HALF), lambda r, i: (i, 0)),
            pl.BlockSpec((MBLK, HALF), lambda r, i: (i, 0)),
            pl.BlockSpec((1, HID), lambda r, i: (0, 0)),
            pl.BlockSpec((1, HID, HID), lambda r, i: (r, 0, 0)),
        ],
        out_specs=pl.BlockSpec((1, MBLK, HID), lambda r, i: (r, i, 0)),
        out_shape=jax.ShapeDtypeStruct((NUM_RELS, m, HID), jnp.float32),
    )(a0, a1, h_bias.reshape(1, HID), weight)


def _out_body(a0_ref, a1_ref, hb_ref, w_ref, b_ref, o_ref):
    x = jnp.concatenate([a0_ref[...], a1_ref[...]], axis=-1)
    x = jnp.maximum(x + hb_ref[...], 0.0)
    logits = jnp.dot(x, w_ref[...], preferred_element_type=jnp.float32)
    logits = logits + b_ref[...]
    mx = jnp.max(logits, axis=-1, keepdims=True)
    e = jnp.exp(logits - mx)
    o_ref[...] = e / jnp.sum(e, axis=-1, keepdims=True)


def _out_proj(a0, a1, h_bias, W_out, b_out):
    m = a0.shape[0]
    return pl.pallas_call(
        _out_body,
        grid=(m // MBLK,),
        in_specs=[
            pl.BlockSpec((MBLK, HALF), lambda i: (i, 0)),
            pl.BlockSpec((MBLK, HALF), lambda i: (i, 0)),
            pl.BlockSpec((1, HID), lambda i: (0, 0)),
            pl.BlockSpec((HID, OUT_DIM), lambda i: (0, 0)),
            pl.BlockSpec((1, OUT_DIM), lambda i: (0, 0)),
        ],
        out_specs=pl.BlockSpec((MBLK, OUT_DIM), lambda i: (i, 0)),
        out_shape=jax.ShapeDtypeStruct((m, OUT_DIM), jnp.float32),
    )(a0, a1, h_bias.reshape(1, HID), W_out, b_out.reshape(1, OUT_DIM))


def _sc_body(table, base_idx, dstg, nrm, out,
             dstS, gidxS, nrmS, pd, pg, pn, gbuf, acc, ssem, gsem):
    c = lax.axis_index("c")
    s = lax.axis_index("s")

    zero16i = jnp.zeros((16,), jnp.int32)
    zero16f = jnp.zeros((16,), jnp.float32)

    tile_lo = s * RPT

    def stage_start(ch, slot):
        sb = pl.ds(slot * CH, CH)
        pltpu.async_copy(dstg.at[pl.ds(ch * CH, CH)], dstS.at[sb], ssem)
        pltpu.async_copy(base_idx.at[pl.ds(ch * CH, CH)], gidxS.at[sb], ssem)
        pltpu.async_copy(nrm.at[pl.ds(ch * CH, CH)], nrmS.at[sb], ssem)

    def stage_wait(slot):
        sb = pl.ds(slot * CH, CH)
        pltpu.make_async_copy(dstg.at[pl.ds(0, CH)], dstS.at[sb], ssem).wait()
        pltpu.make_async_copy(base_idx.at[pl.ds(0, CH)], gidxS.at[sb],
                              ssem).wait()
        pltpu.make_async_copy(nrm.at[pl.ds(0, CH)], nrmS.at[sb], ssem).wait()

    def gather_start(j, slot):
        pltpu.async_copy(table.at[pg.at[pl.ds(j * KBLK, KBLK)]],
                         gbuf.at[pl.ds(slot * KBLK, KBLK)], gsem)

    def gather_wait(slot):
        pltpu.make_async_copy(table.at[pl.ds(0, KBLK)],
                              gbuf.at[pl.ds(slot * KBLK, KBLK)], gsem).wait()

    def proc(j, _, unrolled=True):
        o = j * KBLK
        slot = j % 2
        gather_wait(slot)
        gb = slot * KBLK

        if unrolled:
            def sgrp(g, _2):
                og = o + g * 16
                nv = pn[pl.ds(og, 16)]
                dv = pd[pl.ds(og, 16)]
                for t in range(16):
                    bv = jnp.full((16,), nv[t], dtype=jnp.float32)
                    drow = dv[t]
                    row = gb + g * 16 + t
                    for k in range(HALF // 16):
                        plsc.addupdate(acc.at[drow, pl.ds(k * 16, 16)],
                                       gbuf[row, pl.ds(k * 16, 16)] * bv)
                return 0

            lax.fori_loop(0, KBLK // 16, sgrp, 0)
        else:
            def srow(t, _2):
                nv = pn[pl.ds(o + t, 16)]
                dv = pd[pl.ds(o + t, 16)]
                bv = jnp.full((16,), nv[0], dtype=jnp.float32)
                drow = dv[0]
                for k in range(HALF // 16):
                    plsc.addupdate(acc.at[drow, pl.ds(k * 16, 16)],
                                   gbuf[gb + t, pl.ds(k * 16, 16)] * bv)
                return 0

            lax.fori_loop(0, KBLK, srow, 0)
        return 0

    for p in range(2):
        lo = tile_lo + p * ACC_ROWS
        hi = lo + ACC_ROWS

        def zrow(i, _):
            for k in range(HALF // 16):
                acc[i, pl.ds(k * 16, 16)] = zero16f
            return 0

        lax.fori_loop(0, ACC_ROWS, zrow, 0)

        stage_start(0, 0)

        def chunk(ch, cnt):
            slot = ch % 2
            stage_wait(slot)

            @pl.when(ch + 1 < NCHUNK)
            def _():
                stage_start(ch + 1, 1 - slot)

            def comp(i, cn):
                dvv = dstS[pl.ds(slot * CH + i * 16, 16)]
                gvv = gidxS[pl.ds(slot * CH + i * 16, 16)]
                nvv = nrmS[pl.ds(slot * CH + i * 16, 16)]
                m = (dvv >= lo) & (dvv < hi)
                delta = plsc.all_reduce_population_count(m)[0]
                plsc.store_compressed(pd.at[pl.ds(cn, 16)], dvv - lo, mask=m)
                plsc.store_compressed(pg.at[pl.ds(cn, 16)], gvv + c, mask=m)
                plsc.store_compressed(pn.at[pl.ds(cn, 16)], nvv, mask=m)
                return cn + delta

            cnt = lax.fori_loop(0, CH // 16, comp, cnt)
            nb = cnt // KBLK

            @pl.when(nb > 0)
            def _():
                gather_start(0, 0)

            @pl.when(nb > 1)
            def _():
                gather_start(1, 1)

            def proc2(j, _):
                proc(j, 0)

                @pl.when(j + 2 < nb)
                def _():
                    gather_start(j + 2, j % 2)
                return 0

            lax.fori_loop(0, nb, proc2, 0)
            off = nb * KBLK
            for k in range(KBLK // 16):
                vd = pd[pl.ds(off + k * 16, 16)]
                vg = pg[pl.ds(off + k * 16, 16)]
                vn = pn[pl.ds(off + k * 16, 16)]
                pd[pl.ds(k * 16, 16)] = vd
                pg[pl.ds(k * 16, 16)] = vg
                pn[pl.ds(k * 16, 16)] = vn
            return cnt - off

        cnt = lax.fori_loop(0, NCHUNK, chunk, jnp.int32(0))

        for k in range(KBLK // 16):
            pd[pl.ds(cnt + k * 16, 16)] = zero16i
            pg[pl.ds(cnt + k * 16, 16)] = zero16i
            pn[pl.ds(cnt + k * 16, 16)] = zero16f

        @pl.when(cnt > 0)
        def _():
            gather_start(0, 0)
            proc(0, 0, unrolled=False)

        obase = c * N2 + lo
        pltpu.sync_copy(acc.at[pl.ds(0, ACC_ROWS)],
                        out.at[pl.ds(obase, ACC_ROWS)])


def _sc_aggregate(table_flat, base_idx, dstv, nrmv):
    mesh = plsc.VectorSubcoreMesh(core_axis_name="c", subcore_axis_name="s",
                                  num_cores=NC, num_subcores=NS)
    f = pl.kernel(
        _sc_body,
        out_type=jax.ShapeDtypeStruct((NC * N2, HALF), jnp.float32),
        mesh=mesh,
        compiler_params=pltpu.CompilerParams(needs_layout_passes=False),
        scratch_types=[
            pltpu.VMEM((2 * CH,), jnp.int32),
            pltpu.VMEM((2 * CH,), jnp.int32),
            pltpu.VMEM((2 * CH,), jnp.float32),
            pltpu.VMEM((PBUF,), jnp.int32),
            pltpu.VMEM((PBUF,), jnp.int32),
            pltpu.VMEM((PBUF,), jnp.float32),
            pltpu.VMEM((2 * KBLK, HALF), jnp.float32),
            pltpu.VMEM((ACC_ROWS, HALF), jnp.float32),
            pltpu.SemaphoreType.DMA,
            pltpu.SemaphoreType.DMA,
        ],
    )
    return f(table_flat, base_idx, dstv, nrmv).reshape(NC, N2, HALF)[:, :N, :]


def kernel(g, h, r, norm, W_fnn, b_fnn, basis, coef, h_bias, W_out, b_out):
    src = g[0].astype(jnp.int32)
    dst = g[1].astype(jnp.int32)
    rr = r.reshape(-1).astype(jnp.int32)
    nrmv = norm.reshape(-1)
    base_idx = rr * (2 * N) + src * 2

    weight = _weight_combine(coef, basis)
    x1 = _fnn(h, W_fnn, b_fnn)

    Hr = _rel_matmul(x1, weight[:, :256, :])
    agg = _sc_aggregate(Hr.reshape(2 * NUM_RELS * N, HALF), base_idx, dst, nrmv)

    Hr = _rel_matmul_fused(agg[0], agg[1], h_bias, weight)
    agg = _sc_aggregate(Hr.reshape(2 * NUM_RELS * N, HALF), base_idx, dst, nrmv)

    return _out_proj(agg[0], agg[1], h_bias, W_out, b_out)
